# baseline (speedup 1.0000x reference)
import functools

import jax
import jax.numpy as jnp
from jax.experimental import pallas as pl
from jax.experimental.pallas import tpu as pltpu
from jax.experimental.pallas import tpu_sc as plsc

_B, _P = 4, 2048
_KN = 64
_NC, _NS = 2, 16
_NW = _NC * _NS


def _fps_kernel(n_samples, S, bn, pos_ref, psm_ref, cps_ref):
    iota = (jax.lax.broadcasted_iota(jnp.int32, (S, 128), 0) * 128
            + jax.lax.broadcasted_iota(jnp.int32, (S, 128), 1))
    big = jnp.int32(2 ** 30)
    planes = [(pos_ref[b, 0], pos_ref[b, 1], pos_ref[b, 2]) for b in range(bn)]

    def body(i, carry):
        new = []
        for b in range(bn):
            min_d, cur = carry[2 * b], carry[2 * b + 1]
            px, py, pz = planes[b]
            base = (b * S * 128 + cur) * 3
            sx = psm_ref[base]
            sy = psm_ref[base + 1]
            sz = psm_ref[base + 2]
            cps_ref[b, i * 3] = sx
            cps_ref[b, i * 3 + 1] = sy
            cps_ref[b, i * 3 + 2] = sz
            dx = px - sx
            dy = py - sy
            dz = pz - sz
            d = dx * dx + dy * dy + dz * dz
            min_d = jnp.minimum(min_d, d)
            m = jnp.max(min_d)
            nxt = jnp.min(jnp.where(min_d == m, iota, big))
            new += [min_d, nxt]
        return tuple(new)

    inf = jnp.full((S, 128), jnp.inf, dtype=jnp.float32)
    init = tuple(v for _ in range(bn) for v in (inf, jnp.int32(0)))
    jax.lax.fori_loop(0, n_samples, body, init)


def _fps_pallas(pos_b, n_samples):
    bn, p, _ = pos_b.shape
    s = p // 128
    pt = pos_b.transpose(0, 2, 1).reshape(bn, 3, s, 128)
    cps = pl.pallas_call(
        functools.partial(_fps_kernel, n_samples, s, bn),
        in_specs=[
            pl.BlockSpec(memory_space=pltpu.VMEM),
            pl.BlockSpec(memory_space=pltpu.SMEM),
        ],
        out_specs=pl.BlockSpec(memory_space=pltpu.SMEM),
        out_shape=jax.ShapeDtypeStruct((bn, n_samples * 3), jnp.float32),
    )(pt, pos_b.reshape(-1))
    return cps.reshape(bn, n_samples, 3)


def _sc_gather(table, idx, n_chunks):
    n, = idx.shape
    d = table.shape[1]
    b_per_w = n // _NW
    chunk = b_per_w // n_chunks
    mesh = plsc.VectorSubcoreMesh(core_axis_name="c", subcore_axis_name="s")

    def body(table_hbm, idx_hbm, out_hbm, idx_v, rows_v, sem):
        wid = jax.lax.axis_index("s") * _NC + jax.lax.axis_index("c")
        base = wid * b_per_w

        @pl.loop(0, n_chunks)
        def _chunk_loop(i):
            off = base + i * chunk
            pltpu.sync_copy(idx_hbm.at[pl.ds(off, chunk)], idx_v)
            pltpu.async_copy(table_hbm.at[idx_v], rows_v, sem).wait()
            pltpu.sync_copy(rows_v, out_hbm.at[pl.ds(off, chunk)])

    return pl.kernel(
        body,
        out_type=jax.ShapeDtypeStruct((n, d), jnp.float32),
        mesh=mesh,
        scratch_types=[
            pltpu.VMEM((chunk,), jnp.int32),
            pltpu.VMEM((chunk, d), jnp.float32),
            pltpu.SemaphoreType.DMA,
        ],
    )(table, idx)


def _proj_kernel(dx, x_ref, p_ref, w_ref, u_ref):
    u_ref[...] = (jnp.dot(x_ref[...], w_ref[:dx],
                          preferred_element_type=jnp.float32)
                  + jnp.dot(p_ref[...], w_ref[dx:],
                            preferred_element_type=jnp.float32))


def _proj_pallas(x2d, pos2d, wpad):
    r, dx = x2d.shape
    dpad = wpad.shape[1]
    return pl.pallas_call(
        functools.partial(_proj_kernel, dx),
        out_shape=jax.ShapeDtypeStruct((r, dpad), jnp.float32),
    )(x2d, pos2d, wpad)


def _sa_tail_kernel(d1, g_ref, c_ref, v_ref, w1p_ref, b1_ref, w2_ref, b2_ref,
                    w3_ref, b3_ref, o_ref):
    m, kk, dpad = g_ref.shape
    g = g_ref[..., :d1]
    corr = b1_ref[...] - jnp.dot(c_ref[...], w1p_ref[...],
                                 preferred_element_type=jnp.float32)
    h = jax.nn.relu(g + corr[:, None, :])
    h = h.reshape(m * kk, d1)
    h = jax.nn.relu(jnp.dot(h, w2_ref[...],
                            preferred_element_type=jnp.float32) + b2_ref[...])
    h = jnp.dot(h, w3_ref[...],
                preferred_element_type=jnp.float32) + b3_ref[...]
    dout = h.shape[-1]
    h = h.reshape(m, kk, dout)
    h = jnp.where(v_ref[...][..., None] > 0, h, -jnp.inf)
    out = jnp.max(h, axis=1)
    o_ref[...] = jnp.where(jnp.isfinite(out), out, 0.0)


def _sa_tail_pallas(g, cpos2d, valid2d, d1, w1p, b1, w2, b2, w3, b3, blk):
    nc, kk, dpad = g.shape
    dout = w3.shape[1]
    grid = nc // blk
    return pl.pallas_call(
        functools.partial(_sa_tail_kernel, d1),
        grid=(grid,),
        in_specs=[
            pl.BlockSpec((blk, kk, dpad), lambda i: (i, 0, 0)),
            pl.BlockSpec((blk, 3), lambda i: (i, 0)),
            pl.BlockSpec((blk, kk), lambda i: (i, 0)),
            pl.BlockSpec((3, d1), lambda i: (0, 0)),
            pl.BlockSpec((d1,), lambda i: (0,)),
            pl.BlockSpec(w2.shape, lambda i: (0, 0)),
            pl.BlockSpec(b2.shape, lambda i: (0,)),
            pl.BlockSpec(w3.shape, lambda i: (0, 0)),
            pl.BlockSpec(b3.shape, lambda i: (0,)),
        ],
        out_specs=pl.BlockSpec((blk, dout), lambda i: (i, 0)),
        out_shape=jax.ShapeDtypeStruct((nc, dout), jnp.float32),
    )(g, cpos2d, valid2d, w1p, b1, w2, b2, w3, b3)


def _head_kernel(bn, npts, x_ref, p_ref, w7_ref, b7_ref, w8_ref, b8_ref,
                 w9_ref, b9_ref, w10_ref, b10_ref, w11_ref, b11_ref, o_ref):
    feat = jnp.concatenate([x_ref[...], p_ref[...]], axis=-1)
    h = jax.nn.relu(jnp.dot(feat, w7_ref[...],
                            preferred_element_type=jnp.float32) + b7_ref[...])
    h = jax.nn.relu(jnp.dot(h, w8_ref[...],
                            preferred_element_type=jnp.float32) + b8_ref[...])
    h = jnp.dot(h, w9_ref[...],
                preferred_element_type=jnp.float32) + b9_ref[...]
    g = jnp.max(h.reshape(bn, npts, -1), axis=1)
    g = jax.nn.relu(jnp.dot(g, w10_ref[...],
                            preferred_element_type=jnp.float32) + b10_ref[...])
    o_ref[...] = jnp.dot(g, w11_ref[...],
                         preferred_element_type=jnp.float32) + b11_ref[...]


def _head_pallas(x2, pos2, w7, b7, w8, b8, w9, b9, w10, b10, w11, b11):
    bn, npts, dx = x2.shape
    return pl.pallas_call(
        functools.partial(_head_kernel, bn, npts),
        out_shape=jax.ShapeDtypeStruct((bn, w11.shape[1]), jnp.float32),
    )(x2.reshape(bn * npts, dx), pos2.reshape(bn * npts, 3),
      w7, b7, w8, b8, w9, b9, w10, b10, w11, b11)


def _sa_stage(x_b, pos_b, n_samples, r, params, n_chunks, blk):
    bn, p, dx = x_b.shape
    cpos = _fps_pallas(pos_b, n_samples)
    kk = min(_KN, p)
    d2 = jnp.sum((cpos[:, :, None, :] - pos_b[:, None, :, :]) ** 2, axis=-1)
    negd, nbr = jax.lax.approx_max_k(-d2, kk, recall_target=1.0)
    valid = ((-negd) <= (r * r)).astype(jnp.float32)
    (w1, b1) = params[0]
    d1 = w1.shape[1]
    dpad = -(-d1 // 128) * 128
    wpad = w1
    if dpad != d1:
        wpad = jnp.concatenate(
            [w1, jnp.zeros((w1.shape[0], dpad - d1), jnp.float32)], axis=1)
    u = _proj_pallas(x_b.reshape(bn * p, dx), pos_b.reshape(bn * p, 3), wpad)
    flat = (nbr + (jnp.arange(bn, dtype=jnp.int32) * p)[:, None, None]).reshape(-1)
    g = _sc_gather(u, flat, n_chunks)
    (w2, b2), (w3, b3) = params[1], params[2]
    out = _sa_tail_pallas(
        g.reshape(bn * n_samples, kk, dpad),
        cpos.reshape(bn * n_samples, 3),
        valid.reshape(bn * n_samples, kk),
        d1, w1[dx:], b1, w2, b2, w3, b3, blk)
    return out.reshape(bn, n_samples, -1), cpos


def kernel(pos, color, batch, w1, b1, w2, b2, w3, b3, w4, b4, w5, b5,
           w6, b6, w7, b7, w8, b8, w9, b9, w10, b10, w11, b11):
    pos_b = pos.reshape(_B, _P, 3)
    x_b = color.reshape(_B, _P, 3)
    x1, pos1 = _sa_stage(x_b, pos_b, _P // 2, 0.2,
                         [(w1, b1), (w2, b2), (w3, b3)],
                         n_chunks=16, blk=128)
    x2, pos2 = _sa_stage(x1, pos1, _P // 8, 0.4,
                         [(w4, b4), (w5, b5), (w6, b6)],
                         n_chunks=4, blk=64)
    return _head_pallas(x2, pos2, w7, b7, w8, b8, w9, b9, w10, b10, w11, b11)


# speedup vs baseline: 1.0548x; 1.0548x over previous
import functools

import jax
import jax.numpy as jnp
from jax.experimental import pallas as pl
from jax.experimental.pallas import tpu as pltpu
from jax.experimental.pallas import tpu_sc as plsc

_B, _P = 4, 2048
_KN = 64
_NC, _NS = 2, 16
_NW = _NC * _NS


def _fps_kernel(n_samples, S, bn, pos_ref, psm_ref, cps_ref):
    iota = (jax.lax.broadcasted_iota(jnp.int32, (S, 128), 0) * 128
            + jax.lax.broadcasted_iota(jnp.int32, (S, 128), 1))
    big = jnp.int32(2 ** 30)
    planes = [(pos_ref[b, 0], pos_ref[b, 1], pos_ref[b, 2]) for b in range(bn)]

    def body(i, carry):
        new = []
        for b in range(bn):
            min_d, cur = carry[2 * b], carry[2 * b + 1]
            px, py, pz = planes[b]
            base = (b * S * 128 + cur) * 3
            sx = psm_ref[base]
            sy = psm_ref[base + 1]
            sz = psm_ref[base + 2]
            cps_ref[b, i * 3] = sx
            cps_ref[b, i * 3 + 1] = sy
            cps_ref[b, i * 3 + 2] = sz
            dx = px - sx
            dy = py - sy
            dz = pz - sz
            d = dx * dx + dy * dy + dz * dz
            min_d = jnp.minimum(min_d, d)
            m = jnp.max(min_d)
            nxt = jnp.min(jnp.where(min_d == m, iota, big))
            new += [min_d, nxt]
        return tuple(new)

    inf = jnp.full((S, 128), jnp.inf, dtype=jnp.float32)
    init = tuple(v for _ in range(bn) for v in (inf, jnp.int32(0)))
    jax.lax.fori_loop(0, n_samples, body, init, unroll=4)


def _fps_pallas(pos_b, n_samples):
    bn, p, _ = pos_b.shape
    s = p // 128
    pt = pos_b.transpose(0, 2, 1).reshape(bn, 3, s, 128)
    cps = pl.pallas_call(
        functools.partial(_fps_kernel, n_samples, s, bn),
        in_specs=[
            pl.BlockSpec(memory_space=pltpu.VMEM),
            pl.BlockSpec(memory_space=pltpu.SMEM),
        ],
        out_specs=pl.BlockSpec(memory_space=pltpu.SMEM),
        out_shape=jax.ShapeDtypeStruct((bn, n_samples * 3), jnp.float32),
    )(pt, pos_b.reshape(-1))
    return cps.reshape(bn, n_samples, 3)


def _sc_gather(table, idx, n_chunks):
    n, = idx.shape
    d = table.shape[1]
    b_per_w = n // _NW
    chunk = b_per_w // n_chunks
    mesh = plsc.VectorSubcoreMesh(core_axis_name="c", subcore_axis_name="s")

    def body(table_hbm, idx_hbm, out_hbm, idx_v, rows_v, sem):
        wid = jax.lax.axis_index("s") * _NC + jax.lax.axis_index("c")
        base = wid * b_per_w

        @pl.loop(0, n_chunks)
        def _chunk_loop(i):
            off = base + i * chunk
            pltpu.sync_copy(idx_hbm.at[pl.ds(off, chunk)], idx_v)
            pltpu.async_copy(table_hbm.at[idx_v], rows_v, sem).wait()
            pltpu.sync_copy(rows_v, out_hbm.at[pl.ds(off, chunk)])

    return pl.kernel(
        body,
        out_type=jax.ShapeDtypeStruct((n, d), jnp.float32),
        mesh=mesh,
        scratch_types=[
            pltpu.VMEM((chunk,), jnp.int32),
            pltpu.VMEM((chunk, d), jnp.float32),
            pltpu.SemaphoreType.DMA,
        ],
    )(table, idx)


def _proj_kernel(dx, x_ref, p_ref, w_ref, u_ref):
    u_ref[...] = (jnp.dot(x_ref[...], w_ref[:dx],
                          preferred_element_type=jnp.float32)
                  + jnp.dot(p_ref[...], w_ref[dx:],
                            preferred_element_type=jnp.float32))


def _proj_pallas(x2d, pos2d, wpad):
    r, dx = x2d.shape
    dpad = wpad.shape[1]
    return pl.pallas_call(
        functools.partial(_proj_kernel, dx),
        out_shape=jax.ShapeDtypeStruct((r, dpad), jnp.float32),
    )(x2d, pos2d, wpad)


def _sa_tail_kernel(d1, g_ref, c_ref, v_ref, w1p_ref, b1_ref, w2_ref, b2_ref,
                    w3_ref, b3_ref, o_ref):
    m, kk, dpad = g_ref.shape
    g = g_ref[..., :d1]
    corr = b1_ref[...] - jnp.dot(c_ref[...], w1p_ref[...],
                                 preferred_element_type=jnp.float32)
    h = jax.nn.relu(g + corr[:, None, :])
    h = h.reshape(m * kk, d1)
    h = jax.nn.relu(jnp.dot(h, w2_ref[...],
                            preferred_element_type=jnp.float32) + b2_ref[...])
    h = jnp.dot(h, w3_ref[...],
                preferred_element_type=jnp.float32) + b3_ref[...]
    dout = h.shape[-1]
    h = h.reshape(m, kk, dout)
    h = jnp.where(v_ref[...][..., None] > 0, h, -jnp.inf)
    out = jnp.max(h, axis=1)
    o_ref[...] = jnp.where(jnp.isfinite(out), out, 0.0)


def _sa_tail_pallas(g, cpos2d, valid2d, d1, w1p, b1, w2, b2, w3, b3, blk):
    nc, kk, dpad = g.shape
    dout = w3.shape[1]
    grid = nc // blk
    return pl.pallas_call(
        functools.partial(_sa_tail_kernel, d1),
        grid=(grid,),
        in_specs=[
            pl.BlockSpec((blk, kk, dpad), lambda i: (i, 0, 0)),
            pl.BlockSpec((blk, 3), lambda i: (i, 0)),
            pl.BlockSpec((blk, kk), lambda i: (i, 0)),
            pl.BlockSpec((3, d1), lambda i: (0, 0)),
            pl.BlockSpec((d1,), lambda i: (0,)),
            pl.BlockSpec(w2.shape, lambda i: (0, 0)),
            pl.BlockSpec(b2.shape, lambda i: (0,)),
            pl.BlockSpec(w3.shape, lambda i: (0, 0)),
            pl.BlockSpec(b3.shape, lambda i: (0,)),
        ],
        out_specs=pl.BlockSpec((blk, dout), lambda i: (i, 0)),
        out_shape=jax.ShapeDtypeStruct((nc, dout), jnp.float32),
    )(g, cpos2d, valid2d, w1p, b1, w2, b2, w3, b3)


def _head_kernel(bn, npts, x_ref, p_ref, w7_ref, b7_ref, w8_ref, b8_ref,
                 w9_ref, b9_ref, w10_ref, b10_ref, w11_ref, b11_ref, o_ref):
    feat = jnp.concatenate([x_ref[...], p_ref[...]], axis=-1)
    h = jax.nn.relu(jnp.dot(feat, w7_ref[...],
                            preferred_element_type=jnp.float32) + b7_ref[...])
    h = jax.nn.relu(jnp.dot(h, w8_ref[...],
                            preferred_element_type=jnp.float32) + b8_ref[...])
    h = jnp.dot(h, w9_ref[...],
                preferred_element_type=jnp.float32) + b9_ref[...]
    g = jnp.max(h.reshape(bn, npts, -1), axis=1)
    g = jax.nn.relu(jnp.dot(g, w10_ref[...],
                            preferred_element_type=jnp.float32) + b10_ref[...])
    o_ref[...] = jnp.dot(g, w11_ref[...],
                         preferred_element_type=jnp.float32) + b11_ref[...]


def _head_pallas(x2, pos2, w7, b7, w8, b8, w9, b9, w10, b10, w11, b11):
    bn, npts, dx = x2.shape
    return pl.pallas_call(
        functools.partial(_head_kernel, bn, npts),
        out_shape=jax.ShapeDtypeStruct((bn, w11.shape[1]), jnp.float32),
    )(x2.reshape(bn * npts, dx), pos2.reshape(bn * npts, 3),
      w7, b7, w8, b8, w9, b9, w10, b10, w11, b11)


def _sa_stage(x_b, pos_b, n_samples, r, params, n_chunks, blk):
    bn, p, dx = x_b.shape
    cpos = _fps_pallas(pos_b, n_samples)
    kk = min(_KN, p)
    d2 = jnp.sum((cpos[:, :, None, :] - pos_b[:, None, :, :]) ** 2, axis=-1)
    negd, nbr = jax.lax.approx_max_k(-d2, kk, recall_target=1.0)
    valid = ((-negd) <= (r * r)).astype(jnp.float32)
    (w1, b1) = params[0]
    d1 = w1.shape[1]
    dpad = -(-d1 // 128) * 128
    wpad = w1
    if dpad != d1:
        wpad = jnp.concatenate(
            [w1, jnp.zeros((w1.shape[0], dpad - d1), jnp.float32)], axis=1)
    u = _proj_pallas(x_b.reshape(bn * p, dx), pos_b.reshape(bn * p, 3), wpad)
    flat = (nbr + (jnp.arange(bn, dtype=jnp.int32) * p)[:, None, None]).reshape(-1)
    g = _sc_gather(u, flat, n_chunks)
    (w2, b2), (w3, b3) = params[1], params[2]
    out = _sa_tail_pallas(
        g.reshape(bn * n_samples, kk, dpad),
        cpos.reshape(bn * n_samples, 3),
        valid.reshape(bn * n_samples, kk),
        d1, w1[dx:], b1, w2, b2, w3, b3, blk)
    return out.reshape(bn, n_samples, -1), cpos


def kernel(pos, color, batch, w1, b1, w2, b2, w3, b3, w4, b4, w5, b5,
           w6, b6, w7, b7, w8, b8, w9, b9, w10, b10, w11, b11):
    pos_b = pos.reshape(_B, _P, 3)
    x_b = color.reshape(_B, _P, 3)
    x1, pos1 = _sa_stage(x_b, pos_b, _P // 2, 0.2,
                         [(w1, b1), (w2, b2), (w3, b3)],
                         n_chunks=16, blk=128)
    x2, pos2 = _sa_stage(x1, pos1, _P // 8, 0.4,
                         [(w4, b4), (w5, b5), (w6, b6)],
                         n_chunks=4, blk=64)
    return _head_pallas(x2, pos2, w7, b7, w8, b8, w9, b9, w10, b10, w11, b11)
